# baseline (speedup 1.0000x reference)
import functools

import jax
import jax.numpy as jnp
from jax.experimental import pallas as pl

TOP_K = 100
LANE = 128

def _topk_block_kernel(x_ref, wt_ref, words_ref, scores_ref, *, k):
    _BIG_I32 = jnp.int32(2**30)
    x = x_ref[...]
    wt = wt_ref[...]
    rows, vp = x.shape
    col_iota = jax.lax.broadcasted_iota(jnp.int32, x.shape, 1)
    out_iota = jax.lax.broadcasted_iota(jnp.int32, (rows, LANE), 1)

    def step(j, carry):
        x, words_acc, scores_acc = carry
        m = jnp.max(x, axis=1, keepdims=True)
        eq = x == m
        key = jnp.where(eq, col_iota, _BIG_I32)
        idx = jnp.min(key, axis=1, keepdims=True)
        sel = key == idx
        w = jnp.min(jnp.where(sel, wt, _BIG_I32), axis=1, keepdims=True)
        scores_acc = jnp.where(out_iota == j, m, scores_acc)
        words_acc = jnp.where(out_iota == j, w, words_acc)
        x = jnp.where(sel, -jnp.inf, x)
        return x, words_acc, scores_acc

    init = (
        x,
        jnp.zeros((rows, LANE), jnp.int32),
        jnp.zeros((rows, LANE), jnp.float32),
    )
    _, words_acc, scores_acc = jax.lax.fori_loop(0, k, step, init)
    words_ref[...] = words_acc
    scores_ref[...] = scores_acc


def kernel(y_pred, word_table):
    batch, vocab = y_pred.shape
    vp = ((vocab + LANE - 1) // LANE) * LANE
    rows = 8
    x = jnp.pad(y_pred, ((0, 0), (0, vp - vocab)), constant_values=-jnp.inf)
    wt = jnp.pad(word_table, (0, vp - vocab)).reshape(1, vp)

    words, scores = pl.pallas_call(
        functools.partial(_topk_block_kernel, k=TOP_K),
        grid=(batch // rows,),
        in_specs=[
            pl.BlockSpec((rows, vp), lambda i: (i, 0)),
            pl.BlockSpec((1, vp), lambda i: (0, 0)),
        ],
        out_specs=[
            pl.BlockSpec((rows, LANE), lambda i: (i, 0)),
            pl.BlockSpec((rows, LANE), lambda i: (i, 0)),
        ],
        out_shape=[
            jax.ShapeDtypeStruct((batch, LANE), jnp.int32),
            jax.ShapeDtypeStruct((batch, LANE), jnp.float32),
        ],
    )(x, wt)
    return words[:, :TOP_K], scores[:, :TOP_K]
